# baseline (speedup 1.0000x reference)
import functools

import jax
import jax.numpy as jnp
from jax import lax
from jax.experimental import pallas as pl
from jax.experimental.pallas import tpu as pltpu
from jax.experimental.pallas import tpu_sc as plsc

VOCAB = 100000
DIM = 128
NCLS = 1000
B = 16384

NC = 2
NS = 16
NW = NC * NS
CH = 128
B_PER_W = B // NW
NCHUNK = B_PER_W // CH


def _gather_body(x_hbm, table_hbm, out_hbm, idx_v, rows_v, isem, gsem, wsem):
    wid = lax.axis_index("s") * NC + lax.axis_index("c")
    base = wid * B_PER_W
    icopies = [
        pltpu.async_copy(
            x_hbm.at[pl.ds(base + j * CH, CH)], idx_v.at[j], isem
        )
        for j in range(NCHUNK)
    ]
    streams = []
    for j in range(NCHUNK):
        icopies[j].wait()
        streams.append(
            pltpu.async_copy(
                table_hbm.at[idx_v.at[j]],
                rows_v.at[pl.ds(j * CH, CH)],
                gsem,
            )
        )
    writes = []
    for j in range(NCHUNK):
        streams[j].wait()
        writes.append(
            pltpu.async_copy(
                rows_v.at[pl.ds(j * CH, CH)],
                out_hbm.at[pl.ds(base + j * CH, CH)],
                wsem,
            )
        )
    for cp in writes:
        cp.wait()


_gather = functools.partial(
    pl.kernel,
    mesh=plsc.VectorSubcoreMesh(core_axis_name="c", subcore_axis_name="s"),
    out_type=jax.ShapeDtypeStruct((B, DIM), jnp.float32),
    scratch_types=[
        pltpu.VMEM((NCHUNK, CH), jnp.int32),
        pltpu.VMEM((B_PER_W, DIM), jnp.float32),
        pltpu.SemaphoreType.DMA,
        pltpu.SemaphoreType.DMA,
        pltpu.SemaphoreType.DMA,
    ],
)(_gather_body)


BM = 4096
NSTEPS = B // BM
NQ = 8
STRIPE = BM // NQ


def _out_copies(acc, o_hbm, step, sems):
    row = step * BM
    return [
        pltpu.make_async_copy(
            acc.at[pl.ds(q * STRIPE, STRIPE)],
            o_hbm.at[pl.ds(row + q * STRIPE, STRIPE)],
            sems[q],
        )
        for q in range(NQ)
    ]


def _mm_body(a_ref, w_ref, b_ref, o_hbm, acc0, acc1, *sems):
    i = pl.program_id(0)
    sems0, sems1 = sems[:NQ], sems[NQ:]

    def step(acc, qsems):
        @pl.when(i >= 2)
        def _():
            for cp in _out_copies(acc, o_hbm, i - 2, qsems):
                cp.wait()

        acc[...] = (
            lax.dot_general(
                a_ref[...],
                w_ref[...],
                (((1,), (1,)), ((), ())),
                preferred_element_type=jnp.float32,
            )
            + b_ref[...][None, :]
        )
        for cp in _out_copies(acc, o_hbm, i, qsems):
            cp.start()

    @pl.when(i % 2 == 0)
    def _():
        step(acc0, sems0)

    @pl.when(i % 2 == 1)
    def _():
        step(acc1, sems1)

    @pl.when(i == NSTEPS - 1)
    def _():
        accs = (acc0, acc1) if NSTEPS % 2 == 0 else (acc1, acc0)
        sms = (sems0, sems1) if NSTEPS % 2 == 0 else (sems1, sems0)
        for cp in _out_copies(accs[0], o_hbm, NSTEPS - 2, sms[0]):
            cp.wait()
        for cp in _out_copies(accs[1], o_hbm, NSTEPS - 1, sms[1]):
            cp.wait()


def _matmul(a, w, bias):
    ncls = w.shape[0]
    return pl.pallas_call(
        _mm_body,
        grid=(NSTEPS,),
        in_specs=[
            pl.BlockSpec((BM, DIM), lambda i: (i, 0)),
            pl.BlockSpec((ncls, DIM), lambda i: (0, 0)),
            pl.BlockSpec((ncls,), lambda i: (0,)),
        ],
        out_specs=pl.BlockSpec(memory_space=pl.ANY),
        out_shape=jax.ShapeDtypeStruct((B, ncls), jnp.float32),
        scratch_shapes=[
            pltpu.VMEM((BM, ncls), jnp.float32),
            pltpu.VMEM((BM, ncls), jnp.float32),
        ]
        + [pltpu.SemaphoreType.DMA] * (2 * NQ),
    )(a, w, bias)


def kernel(x, offsets, emb_weight, fc_weight, fc_bias):
    del offsets
    gathered = _gather(x, emb_weight)
    return _matmul(gathered, fc_weight, fc_bias)


# speedup vs baseline: 1.0161x; 1.0161x over previous
import functools

import jax
import jax.numpy as jnp
from jax import lax
from jax.experimental import pallas as pl
from jax.experimental.pallas import tpu as pltpu
from jax.experimental.pallas import tpu_sc as plsc

VOCAB = 100000
DIM = 128
NCLS = 1000
B = 16384

NC = 2
NS = 16
NW = NC * NS
CH = 128
B_PER_W = B // NW
NCHUNK = B_PER_W // CH


def _gather_body(x_hbm, table_hbm, out_hbm, idx_v, rows_v, isem, gsem, wsem):
    wid = lax.axis_index("s") * NC + lax.axis_index("c")
    base = wid * B_PER_W
    icopies = [
        pltpu.async_copy(
            x_hbm.at[pl.ds(base + j * CH, CH)], idx_v.at[j], isem
        )
        for j in range(NCHUNK)
    ]
    streams = []
    for j in range(NCHUNK):
        icopies[j].wait()
        streams.append(
            pltpu.async_copy(
                table_hbm.at[idx_v.at[j]],
                rows_v.at[pl.ds(j * CH, CH)],
                gsem,
            )
        )
    writes = []
    for j in range(NCHUNK):
        streams[j].wait()
        writes.append(
            pltpu.async_copy(
                rows_v.at[pl.ds(j * CH, CH)],
                out_hbm.at[pl.ds(base + j * CH, CH)],
                wsem,
            )
        )
    for cp in writes:
        cp.wait()


_gather = functools.partial(
    pl.kernel,
    mesh=plsc.VectorSubcoreMesh(core_axis_name="c", subcore_axis_name="s"),
    out_type=jax.ShapeDtypeStruct((B, DIM), jnp.float32),
    scratch_types=[
        pltpu.VMEM((NCHUNK, CH), jnp.int32),
        pltpu.VMEM((B_PER_W, DIM), jnp.float32),
        pltpu.SemaphoreType.DMA,
        pltpu.SemaphoreType.DMA,
        pltpu.SemaphoreType.DMA,
    ],
)(_gather_body)


BM = 2048
NSTEPS = B // BM
NQ = 4
STRIPE = BM // NQ


def _out_copies(acc, o_hbm, step, sems):
    row = step * BM
    return [
        pltpu.make_async_copy(
            acc.at[pl.ds(q * STRIPE, STRIPE)],
            o_hbm.at[pl.ds(row + q * STRIPE, STRIPE)],
            sems[q],
        )
        for q in range(NQ)
    ]


def _mm_body(a_ref, w_ref, b_ref, o_hbm, acc0, acc1, *sems):
    i = pl.program_id(0)
    sems0, sems1 = sems[:NQ], sems[NQ:]

    def step(acc, qsems):
        @pl.when(i >= 2)
        def _():
            for cp in _out_copies(acc, o_hbm, i - 2, qsems):
                cp.wait()

        acc[...] = (
            lax.dot_general(
                a_ref[...],
                w_ref[...],
                (((1,), (1,)), ((), ())),
                preferred_element_type=jnp.float32,
            )
            + b_ref[...][None, :]
        )
        for cp in _out_copies(acc, o_hbm, i, qsems):
            cp.start()

    @pl.when(i % 2 == 0)
    def _():
        step(acc0, sems0)

    @pl.when(i % 2 == 1)
    def _():
        step(acc1, sems1)

    @pl.when(i == NSTEPS - 1)
    def _():
        accs = (acc0, acc1) if NSTEPS % 2 == 0 else (acc1, acc0)
        sms = (sems0, sems1) if NSTEPS % 2 == 0 else (sems1, sems0)
        for cp in _out_copies(accs[0], o_hbm, NSTEPS - 2, sms[0]):
            cp.wait()
        for cp in _out_copies(accs[1], o_hbm, NSTEPS - 1, sms[1]):
            cp.wait()


def _matmul(a, w, bias):
    ncls = w.shape[0]
    return pl.pallas_call(
        _mm_body,
        grid=(NSTEPS,),
        in_specs=[
            pl.BlockSpec((BM, DIM), lambda i: (i, 0)),
            pl.BlockSpec((ncls, DIM), lambda i: (0, 0)),
            pl.BlockSpec((ncls,), lambda i: (0,)),
        ],
        out_specs=pl.BlockSpec(memory_space=pl.ANY),
        out_shape=jax.ShapeDtypeStruct((B, ncls), jnp.float32),
        scratch_shapes=[
            pltpu.VMEM((BM, ncls), jnp.float32),
            pltpu.VMEM((BM, ncls), jnp.float32),
        ]
        + [pltpu.SemaphoreType.DMA] * (2 * NQ),
    )(a, w, bias)


def kernel(x, offsets, emb_weight, fc_weight, fc_bias):
    del offsets
    gathered = _gather(x, emb_weight)
    return _matmul(gathered, fc_weight, fc_bias)


# speedup vs baseline: 1.0173x; 1.0011x over previous
import functools

import jax
import jax.numpy as jnp
from jax import lax
from jax.experimental import pallas as pl
from jax.experimental.pallas import tpu as pltpu
from jax.experimental.pallas import tpu_sc as plsc

VOCAB = 100000
DIM = 128
NCLS = 1000
B = 16384

NC = 2
NS = 16
NW = NC * NS
CH = 64
B_PER_W = B // NW
NCHUNK = B_PER_W // CH


def _gather_body(x_hbm, table_hbm, out_hbm, idx_v, rows_v, isem, gsem, wsem):
    wid = lax.axis_index("s") * NC + lax.axis_index("c")
    base = wid * B_PER_W
    icopies = [
        pltpu.async_copy(
            x_hbm.at[pl.ds(base + j * CH, CH)], idx_v.at[j], isem
        )
        for j in range(NCHUNK)
    ]
    streams = []
    for j in range(NCHUNK):
        icopies[j].wait()
        streams.append(
            pltpu.async_copy(
                table_hbm.at[idx_v.at[j]],
                rows_v.at[pl.ds(j * CH, CH)],
                gsem,
            )
        )
    writes = []
    for j in range(NCHUNK):
        streams[j].wait()
        writes.append(
            pltpu.async_copy(
                rows_v.at[pl.ds(j * CH, CH)],
                out_hbm.at[pl.ds(base + j * CH, CH)],
                wsem,
            )
        )
    for cp in writes:
        cp.wait()


_gather = functools.partial(
    pl.kernel,
    mesh=plsc.VectorSubcoreMesh(core_axis_name="c", subcore_axis_name="s"),
    out_type=jax.ShapeDtypeStruct((B, DIM), jnp.float32),
    scratch_types=[
        pltpu.VMEM((NCHUNK, CH), jnp.int32),
        pltpu.VMEM((B_PER_W, DIM), jnp.float32),
        pltpu.SemaphoreType.DMA,
        pltpu.SemaphoreType.DMA,
        pltpu.SemaphoreType.DMA,
    ],
)(_gather_body)


BM = 2048
NSTEPS = B // BM
NQ = 4
STRIPE = BM // NQ


def _out_copies(acc, o_hbm, step, sems):
    row = step * BM
    return [
        pltpu.make_async_copy(
            acc.at[pl.ds(q * STRIPE, STRIPE)],
            o_hbm.at[pl.ds(row + q * STRIPE, STRIPE)],
            sems[q],
        )
        for q in range(NQ)
    ]


def _mm_body(a_ref, w_ref, b_ref, o_hbm, acc0, acc1, *sems):
    i = pl.program_id(0)
    sems0, sems1 = sems[:NQ], sems[NQ:]

    def step(acc, qsems):
        @pl.when(i >= 2)
        def _():
            for cp in _out_copies(acc, o_hbm, i - 2, qsems):
                cp.wait()

        acc[...] = (
            lax.dot_general(
                a_ref[...],
                w_ref[...],
                (((1,), (1,)), ((), ())),
                preferred_element_type=jnp.float32,
            )
            + b_ref[...][None, :]
        )
        for cp in _out_copies(acc, o_hbm, i, qsems):
            cp.start()

    @pl.when(i % 2 == 0)
    def _():
        step(acc0, sems0)

    @pl.when(i % 2 == 1)
    def _():
        step(acc1, sems1)

    @pl.when(i == NSTEPS - 1)
    def _():
        accs = (acc0, acc1) if NSTEPS % 2 == 0 else (acc1, acc0)
        sms = (sems0, sems1) if NSTEPS % 2 == 0 else (sems1, sems0)
        for cp in _out_copies(accs[0], o_hbm, NSTEPS - 2, sms[0]):
            cp.wait()
        for cp in _out_copies(accs[1], o_hbm, NSTEPS - 1, sms[1]):
            cp.wait()


def _matmul(a, w, bias):
    ncls = w.shape[0]
    return pl.pallas_call(
        _mm_body,
        grid=(NSTEPS,),
        in_specs=[
            pl.BlockSpec((BM, DIM), lambda i: (i, 0)),
            pl.BlockSpec((ncls, DIM), lambda i: (0, 0)),
            pl.BlockSpec((ncls,), lambda i: (0,)),
        ],
        out_specs=pl.BlockSpec(memory_space=pl.ANY),
        out_shape=jax.ShapeDtypeStruct((B, ncls), jnp.float32),
        scratch_shapes=[
            pltpu.VMEM((BM, ncls), jnp.float32),
            pltpu.VMEM((BM, ncls), jnp.float32),
        ]
        + [pltpu.SemaphoreType.DMA] * (2 * NQ),
    )(a, w, bias)


def kernel(x, offsets, emb_weight, fc_weight, fc_bias):
    del offsets
    gathered = _gather(x, emb_weight)
    return _matmul(gathered, fc_weight, fc_bias)
